# baseline (speedup 1.0000x reference)
import functools

import jax
import jax.numpy as jnp
from jax import lax
from jax.experimental import pallas as pl
from jax.experimental.pallas import tpu as pltpu
from jax.experimental.pallas import tpu_sc as plsc

_NC = 2
_NS = 16
_L = 16
_NW = _NC * _NS

_N = 10000
_D = 128
_D2 = _D // 2
_E = 320000
_EPW = _E // _NW
_CHUNK = 128
_NCHUNK = -(-_EPW // _CHUNK)
_LAST_OFF = _EPW - _CHUNK
_INV_TEMP = 10.0


def _normalize_body(a_ref, b_ref, pa_ref, pb_ref):
    for src, dst in ((a_ref, pa_ref), (b_ref, pb_ref)):
        x = src[...]
        norm = jnp.sqrt(jnp.sum(x * x, axis=-1, keepdims=True))
        y = (x / jnp.maximum(norm, 1e-12)).astype(jnp.bfloat16)
        lo = lax.bitcast_convert_type(y[:, :_D2], jnp.uint16).astype(jnp.uint32)
        hi = lax.bitcast_convert_type(y[:, _D2:], jnp.uint16).astype(jnp.uint32)
        dst[...] = lo | (hi << 16)


_NBLK = 1000


def _normalize(userA, userB):
    return pl.pallas_call(
        _normalize_body,
        grid=(_N // _NBLK,),
        in_specs=[pl.BlockSpec((_NBLK, _D), lambda i: (i, 0))] * 2,
        out_specs=[pl.BlockSpec((_NBLK, _D2), lambda i: (i, 0))] * 2,
        out_shape=(
            jax.ShapeDtypeStruct((_N, _D2), jnp.uint32),
            jax.ShapeDtypeStruct((_N, _D2), jnp.uint32),
        ),
    )(userA, userB)


def _sc_body(tabA, tabB, idx_hbm, out_hbm,
             idxA_v, idxB_v, rowsA0, rowsB0, rowsA1, rowsB1, out_v,
             sem0, sem1):
    sid = lax.axis_index("s")
    wid = sid * _NC + lax.axis_index("c")
    base = pl.multiple_of(wid * _EPW, 8)
    pltpu.sync_copy(idx_hbm.at[0, pl.ds(base, _EPW)], idxA_v)
    pltpu.sync_copy(idx_hbm.at[1, pl.ds(base, _EPW)], idxB_v)
    lanes = lax.iota(jnp.int32, _L)
    bufs = ((rowsA0, rowsB0, sem0), (rowsA1, rowsB1, sem1))

    def chunk_off(cc):
        return pl.multiple_of(jnp.minimum(cc * _CHUNK, _LAST_OFF), 8)

    def issue(cc, ra, rb, sem):
        off = chunk_off(cc)
        pltpu.async_copy(tabA.at[idxA_v.at[pl.ds(off, _CHUNK)]], ra, sem)
        pltpu.async_copy(tabB.at[idxB_v.at[pl.ds(off, _CHUNK)]], rb, sem)

    def drain(ra, rb, sem):
        pltpu.make_async_copy(tabA.at[idxA_v.at[pl.ds(0, _CHUNK)]],
                              ra, sem).wait()
        pltpu.make_async_copy(tabB.at[idxB_v.at[pl.ds(0, _CHUNK)]],
                              rb, sem).wait()

    def compute(c, rowsA_v, rowsB_v):
        off = chunk_off(c)

        def ebody(t, _):
            res = jnp.zeros((_L,), jnp.float32)
            for u in range(_L):
                e = t * _L + u
                ps = []
                for q in range(_D2 // _L):
                    a = plsc.bitcast(rowsA_v[e, pl.ds(q * _L, _L)],
                                     jnp.bfloat16)
                    b = plsc.bitcast(rowsB_v[e, pl.ds(q * _L, _L)],
                                     jnp.bfloat16)
                    ps.append(a * b)
                s = (ps[0] + ps[1]) + (ps[2] + ps[3])
                plo, phi = plsc.unpack(s, format=plsc.PackFormat.INTERLEAVED)
                res = jnp.where(lanes == u, jnp.sum(plo + phi), res)
            out_v[pl.ds(pl.multiple_of(t * _L, 8), _L)] = res * _INV_TEMP
            return 0

        lax.fori_loop(0, _CHUNK // _L, ebody, 0)
        pltpu.sync_copy(out_v, out_hbm.at[pl.ds(base + off, _CHUNK)])

    issue(0, *bufs[0])
    issue(1, *bufs[1])

    def pair_body(p, _):
        c = p * 2
        for b in range(2):
            ra, rb, sem = bufs[b]
            cc = c + b

            @pl.when(cc < _NCHUNK)
            def _process():
                drain(ra, rb, sem)
                compute(cc, ra, rb)

                @pl.when(cc + 2 < _NCHUNK)
                def _prefetch():
                    issue(cc + 2, ra, rb, sem)

        return 0

    lax.fori_loop(0, (_NCHUNK + 1) // 2, pair_body, 0)


_sc_call = functools.partial(
    pl.kernel,
    out_type=jax.ShapeDtypeStruct((_E,), jnp.float32),
    mesh=plsc.VectorSubcoreMesh(core_axis_name="c", subcore_axis_name="s"),
    compiler_params=pltpu.CompilerParams(needs_layout_passes=False,
                                         use_tc_tiling_on_sc=False),
    scratch_types=[
        pltpu.VMEM((_EPW,), jnp.int32),
        pltpu.VMEM((_EPW,), jnp.int32),
        pltpu.VMEM((_CHUNK, _D2), jnp.uint32),
        pltpu.VMEM((_CHUNK, _D2), jnp.uint32),
        pltpu.VMEM((_CHUNK, _D2), jnp.uint32),
        pltpu.VMEM((_CHUNK, _D2), jnp.uint32),
        pltpu.VMEM((_CHUNK,), jnp.float32),
        pltpu.SemaphoreType.DMA,
        pltpu.SemaphoreType.DMA,
    ],
)(_sc_body)


def kernel(userA, userB, edge_label_index):
    tabA, tabB = _normalize(userA, userB)
    idx = edge_label_index.astype(jnp.int32)
    return _sc_call(tabA, tabB, idx)


# speedup vs baseline: 1.0132x; 1.0132x over previous
import functools

import jax
import jax.numpy as jnp
from jax import lax
from jax.experimental import pallas as pl
from jax.experimental.pallas import tpu as pltpu
from jax.experimental.pallas import tpu_sc as plsc

_NC = 2
_NS = 16
_L = 16
_NW = _NC * _NS

_N = 10000
_D = 128
_D2 = _D // 2
_E = 320000
_EPW = _E // _NW
_CHUNK = 128
_NCHUNK = -(-_EPW // _CHUNK)
_LAST_OFF = _EPW - _CHUNK
_INV_TEMP = 10.0


def _normalize_body(a_ref, b_ref, pa_ref, pb_ref):
    for src, dst in ((a_ref, pa_ref), (b_ref, pb_ref)):
        x = src[...]
        norm = jnp.sqrt(jnp.sum(x * x, axis=-1, keepdims=True))
        y = (x / jnp.maximum(norm, 1e-12)).astype(jnp.bfloat16)
        lo = lax.bitcast_convert_type(y[:, :_D2], jnp.uint16).astype(jnp.uint32)
        hi = lax.bitcast_convert_type(y[:, _D2:], jnp.uint16).astype(jnp.uint32)
        dst[...] = lo | (hi << 16)


_NBLK = 1000


def _normalize(userA, userB):
    return pl.pallas_call(
        _normalize_body,
        grid=(_N // _NBLK,),
        in_specs=[pl.BlockSpec((_NBLK, _D), lambda i: (i, 0))] * 2,
        out_specs=[pl.BlockSpec((_NBLK, _D2), lambda i: (i, 0))] * 2,
        out_shape=(
            jax.ShapeDtypeStruct((_N, _D2), jnp.uint32),
            jax.ShapeDtypeStruct((_N, _D2), jnp.uint32),
        ),
    )(userA, userB)


def _sc_body(tabA, tabB, idx_hbm, out_hbm,
             idxA_v, idxB_v, rowsA0, rowsB0, rowsA1, rowsB1, out0, out1,
             sem0, sem1, osem0, osem1):
    sid = lax.axis_index("s")
    wid = sid * _NC + lax.axis_index("c")
    base = pl.multiple_of(wid * _EPW, 8)
    pltpu.sync_copy(idx_hbm.at[0, pl.ds(base, _EPW)], idxA_v)
    pltpu.sync_copy(idx_hbm.at[1, pl.ds(base, _EPW)], idxB_v)
    lanes = lax.iota(jnp.int32, _L)
    bufs = ((rowsA0, rowsB0, out0, sem0, osem0),
            (rowsA1, rowsB1, out1, sem1, osem1))

    def chunk_off(cc):
        return pl.multiple_of(jnp.minimum(cc * _CHUNK, _LAST_OFF), 8)

    def issue(cc, ra, rb, ov, sem, osem):
        del ov, osem
        off = chunk_off(cc)
        pltpu.async_copy(tabA.at[idxA_v.at[pl.ds(off, _CHUNK)]], ra, sem)
        pltpu.async_copy(tabB.at[idxB_v.at[pl.ds(off, _CHUNK)]], rb, sem)

    def drain(ra, rb, sem):
        pltpu.make_async_copy(tabA.at[idxA_v.at[pl.ds(0, _CHUNK)]],
                              ra, sem).wait()
        pltpu.make_async_copy(tabB.at[idxB_v.at[pl.ds(0, _CHUNK)]],
                              rb, sem).wait()

    def compute(c, rowsA_v, rowsB_v, out_v, osem):
        off = chunk_off(c)

        @pl.when(c >= 2)
        def _drain_store():
            pltpu.make_async_copy(out_v, out_hbm.at[pl.ds(0, _CHUNK)],
                                  osem).wait()

        def ebody(t, _):
            res = jnp.zeros((_L,), jnp.float32)
            for u in range(_L):
                e = t * _L + u
                ps = []
                for q in range(_D2 // _L):
                    a = plsc.bitcast(rowsA_v[e, pl.ds(q * _L, _L)],
                                     jnp.bfloat16)
                    b = plsc.bitcast(rowsB_v[e, pl.ds(q * _L, _L)],
                                     jnp.bfloat16)
                    ps.append(a * b)
                s = (ps[0] + ps[1]) + (ps[2] + ps[3])
                plo, phi = plsc.unpack(s, format=plsc.PackFormat.INTERLEAVED)
                res = jnp.where(lanes == u, jnp.sum(plo + phi), res)
            out_v[pl.ds(pl.multiple_of(t * _L, 8), _L)] = res * _INV_TEMP
            return 0

        lax.fori_loop(0, _CHUNK // _L, ebody, 0)
        pltpu.async_copy(out_v, out_hbm.at[pl.ds(base + off, _CHUNK)], osem)

    issue(0, *bufs[0])
    issue(1, *bufs[1])

    def pair_body(p, _):
        c = p * 2
        for b in range(2):
            ra, rb, ov, sem, osem = bufs[b]
            cc = c + b

            @pl.when(cc < _NCHUNK)
            def _process():
                drain(ra, rb, sem)
                compute(cc, ra, rb, ov, osem)

                @pl.when(cc + 2 < _NCHUNK)
                def _prefetch():
                    issue(cc + 2, *bufs[b])

        return 0

    lax.fori_loop(0, (_NCHUNK + 1) // 2, pair_body, 0)
    for _ra, _rb, ov, _sem, osem in bufs:
        pltpu.make_async_copy(ov, out_hbm.at[pl.ds(0, _CHUNK)], osem).wait()


_sc_call = functools.partial(
    pl.kernel,
    out_type=jax.ShapeDtypeStruct((_E,), jnp.float32),
    mesh=plsc.VectorSubcoreMesh(core_axis_name="c", subcore_axis_name="s"),
    compiler_params=pltpu.CompilerParams(needs_layout_passes=False,
                                         use_tc_tiling_on_sc=False),
    scratch_types=[
        pltpu.VMEM((_EPW,), jnp.int32),
        pltpu.VMEM((_EPW,), jnp.int32),
        pltpu.VMEM((_CHUNK, _D2), jnp.uint32),
        pltpu.VMEM((_CHUNK, _D2), jnp.uint32),
        pltpu.VMEM((_CHUNK, _D2), jnp.uint32),
        pltpu.VMEM((_CHUNK, _D2), jnp.uint32),
        pltpu.VMEM((_CHUNK,), jnp.float32),
        pltpu.VMEM((_CHUNK,), jnp.float32),
        pltpu.SemaphoreType.DMA,
        pltpu.SemaphoreType.DMA,
        pltpu.SemaphoreType.DMA,
        pltpu.SemaphoreType.DMA,
    ],
)(_sc_body)


def kernel(userA, userB, edge_label_index):
    tabA, tabB = _normalize(userA, userB)
    idx = edge_label_index.astype(jnp.int32)
    return _sc_call(tabA, tabB, idx)


# speedup vs baseline: 1.0985x; 1.0842x over previous
import functools

import jax
import jax.numpy as jnp
from jax import lax
from jax.experimental import pallas as pl
from jax.experimental.pallas import tpu as pltpu
from jax.experimental.pallas import tpu_sc as plsc

_NC = 2
_NS = 16
_L = 16
_NW = _NC * _NS

_N = 10000
_D = 128
_D2 = _D // 2
_E = 320000
_EPW = _E // _NW
_CHUNK = 256
_NCHUNK = -(-_EPW // _CHUNK)
_LAST_OFF = _EPW - _CHUNK
_STRM = 128
_INV_TEMP = 10.0


def _normalize_body(a_ref, b_ref, pa_ref, pb_ref):
    for src, dst in ((a_ref, pa_ref), (b_ref, pb_ref)):
        x = src[...]
        norm = jnp.sqrt(jnp.sum(x * x, axis=-1, keepdims=True))
        y = (x / jnp.maximum(norm, 1e-12)).astype(jnp.bfloat16)
        lo = lax.bitcast_convert_type(y[:, :_D2], jnp.uint16).astype(jnp.uint32)
        hi = lax.bitcast_convert_type(y[:, _D2:], jnp.uint16).astype(jnp.uint32)
        dst[...] = lo | (hi << 16)


_NBLK = 1000


def _normalize(userA, userB):
    return pl.pallas_call(
        _normalize_body,
        grid=(_N // _NBLK,),
        in_specs=[pl.BlockSpec((_NBLK, _D), lambda i: (i, 0))] * 2,
        out_specs=[pl.BlockSpec((_NBLK, _D2), lambda i: (i, 0))] * 2,
        out_shape=(
            jax.ShapeDtypeStruct((_N, _D2), jnp.uint32),
            jax.ShapeDtypeStruct((_N, _D2), jnp.uint32),
        ),
    )(userA, userB)


def _sc_body(tabA, tabB, idx_hbm, out_hbm,
             idxA_v, idxB_v, rowsA0, rowsB0, rowsA1, rowsB1, out0, out1,
             sem0, sem1, osem0, osem1):
    sid = lax.axis_index("s")
    wid = sid * _NC + lax.axis_index("c")
    base = pl.multiple_of(wid * _EPW, 8)
    pltpu.sync_copy(idx_hbm.at[0, pl.ds(base, _EPW)], idxA_v)
    pltpu.sync_copy(idx_hbm.at[1, pl.ds(base, _EPW)], idxB_v)
    lanes = lax.iota(jnp.int32, _L)
    bufs = ((rowsA0, rowsB0, out0, sem0, osem0),
            (rowsA1, rowsB1, out1, sem1, osem1))

    def chunk_off(cc):
        return pl.multiple_of(jnp.minimum(cc * _CHUNK, _LAST_OFF), 8)

    def issue(cc, ra, rb, ov, sem, osem):
        del ov, osem
        off = chunk_off(cc)
        for h in range(_CHUNK // _STRM):
            o = pl.multiple_of(off + h * _STRM, 8)
            r = pl.multiple_of(h * _STRM, 8)
            pltpu.async_copy(tabA.at[idxA_v.at[pl.ds(o, _STRM)]],
                             ra.at[pl.ds(r, _STRM)], sem)
            pltpu.async_copy(tabB.at[idxB_v.at[pl.ds(o, _STRM)]],
                             rb.at[pl.ds(r, _STRM)], sem)

    def drain(ra, rb, sem):
        for h in range(_CHUNK // _STRM):
            r = pl.multiple_of(h * _STRM, 8)
            pltpu.make_async_copy(tabA.at[idxA_v.at[pl.ds(0, _STRM)]],
                                  ra.at[pl.ds(r, _STRM)], sem).wait()
            pltpu.make_async_copy(tabB.at[idxB_v.at[pl.ds(0, _STRM)]],
                                  rb.at[pl.ds(r, _STRM)], sem).wait()

    def compute(c, rowsA_v, rowsB_v, out_v, osem):
        off = chunk_off(c)

        @pl.when(c >= 2)
        def _drain_store():
            pltpu.make_async_copy(out_v, out_hbm.at[pl.ds(0, _CHUNK)],
                                  osem).wait()

        def ebody(t, _):
            res = jnp.zeros((_L,), jnp.float32)
            for u in range(_L):
                e = t * _L + u
                ps = []
                for q in range(_D2 // _L):
                    a = plsc.bitcast(rowsA_v[e, pl.ds(q * _L, _L)],
                                     jnp.bfloat16)
                    b = plsc.bitcast(rowsB_v[e, pl.ds(q * _L, _L)],
                                     jnp.bfloat16)
                    ps.append(a * b)
                s = (ps[0] + ps[1]) + (ps[2] + ps[3])
                plo, phi = plsc.unpack(s, format=plsc.PackFormat.INTERLEAVED)
                res = jnp.where(lanes == u, jnp.sum(plo + phi), res)
            out_v[pl.ds(pl.multiple_of(t * _L, 8), _L)] = res * _INV_TEMP
            return 0

        lax.fori_loop(0, _CHUNK // _L, ebody, 0)
        pltpu.async_copy(out_v, out_hbm.at[pl.ds(base + off, _CHUNK)], osem)

    issue(0, *bufs[0])
    issue(1, *bufs[1])

    def pair_body(p, _):
        c = p * 2
        for b in range(2):
            ra, rb, ov, sem, osem = bufs[b]
            cc = c + b

            @pl.when(cc < _NCHUNK)
            def _process():
                drain(ra, rb, sem)
                compute(cc, ra, rb, ov, osem)

                @pl.when(cc + 2 < _NCHUNK)
                def _prefetch():
                    issue(cc + 2, *bufs[b])

        return 0

    lax.fori_loop(0, (_NCHUNK + 1) // 2, pair_body, 0)
    for _ra, _rb, ov, _sem, osem in bufs:
        pltpu.make_async_copy(ov, out_hbm.at[pl.ds(0, _CHUNK)], osem).wait()


_sc_call = functools.partial(
    pl.kernel,
    out_type=jax.ShapeDtypeStruct((_E,), jnp.float32),
    mesh=plsc.VectorSubcoreMesh(core_axis_name="c", subcore_axis_name="s"),
    compiler_params=pltpu.CompilerParams(needs_layout_passes=False,
                                         use_tc_tiling_on_sc=False),
    scratch_types=[
        pltpu.VMEM((_EPW,), jnp.int32),
        pltpu.VMEM((_EPW,), jnp.int32),
        pltpu.VMEM((_CHUNK, _D2), jnp.uint32),
        pltpu.VMEM((_CHUNK, _D2), jnp.uint32),
        pltpu.VMEM((_CHUNK, _D2), jnp.uint32),
        pltpu.VMEM((_CHUNK, _D2), jnp.uint32),
        pltpu.VMEM((_CHUNK,), jnp.float32),
        pltpu.VMEM((_CHUNK,), jnp.float32),
        pltpu.SemaphoreType.DMA,
        pltpu.SemaphoreType.DMA,
        pltpu.SemaphoreType.DMA,
        pltpu.SemaphoreType.DMA,
    ],
)(_sc_body)


def kernel(userA, userB, edge_label_index):
    tabA, tabB = _normalize(userA, userB)
    idx = edge_label_index.astype(jnp.int32)
    return _sc_call(tabA, tabB, idx)
